# Initial kernel scaffold; baseline (speedup 1.0000x reference)
import jax
import jax.numpy as jnp
from jax.experimental import pallas as pl


def kernel(edge_attr, W0, W1, W2):
    raise NotImplementedError("write your pallas kernel here")


# speedup vs baseline: 1.2558x; 1.2558x over previous
import functools

import jax
import jax.numpy as jnp
from jax import lax
from jax.experimental import pallas as pl
from jax.experimental.pallas import tpu as pltpu
from jax.experimental.pallas import tpu_sc as plsc

HD = 256
T_ROWS = 64
NW = 32
BLK = 128


def _table_body(w0, w1, w2, o):
    for i in range(5):
        for j in range(6):
            for k in range(2):
                r = 12 * i + 2 * j + k
                o[pl.ds(r, 1), :] = (
                    w0[pl.ds(i, 1), :] + w1[pl.ds(j, 1), :] + w2[pl.ds(k, 1), :]
                )
    for r in range(60, T_ROWS):
        o[pl.ds(r, 1), :] = jnp.zeros((1, HD), jnp.float32)


def _build_table(W0, W1, W2):
    return pl.pallas_call(
        _table_body,
        out_shape=jax.ShapeDtypeStruct((T_ROWS, HD), jnp.float32),
    )(W0, W1, W2)


def _sc_body(chunk, a0_hbm, a1_hbm, a2_hbm, t_hbm, out_hbm,
             a0_v, a1_v, a2_v, cidx_v, buf_v, sem):
    wid = lax.axis_index("s") * 2 + lax.axis_index("c")
    base = wid * chunk
    pltpu.sync_copy(a0_hbm.at[pl.ds(base, chunk)], a0_v.at[pl.ds(0, chunk)])
    pltpu.sync_copy(a1_hbm.at[pl.ds(base, chunk)], a1_v.at[pl.ds(0, chunk)])
    pltpu.sync_copy(a2_hbm.at[pl.ds(base, chunk)], a2_v.at[pl.ds(0, chunk)])

    nvec = (chunk + 15) // 16

    def cbody(i, _):
        s = i * 16
        c = a0_v[pl.ds(s, 16)] * 12 + a1_v[pl.ds(s, 16)] * 2 + a2_v[pl.ds(s, 16)]
        cidx_v[pl.ds(s, 16)] = c
        return 0

    lax.fori_loop(0, nvec, cbody, 0)

    nfull = chunk // BLK
    tail = chunk - nfull * BLK

    def gbody(j, _):
        s = j * BLK
        pltpu.async_copy(
            t_hbm.at[cidx_v.at[pl.ds(s, BLK)]], buf_v, sem
        ).wait()
        pltpu.sync_copy(buf_v, out_hbm.at[pl.ds(base + s, BLK)])
        return 0

    lax.fori_loop(0, nfull, gbody, 0)

    if tail:
        s = nfull * BLK
        pltpu.async_copy(
            t_hbm.at[cidx_v.at[pl.ds(s, tail)]], buf_v.at[pl.ds(0, tail)], sem
        ).wait()
        pltpu.sync_copy(buf_v.at[pl.ds(0, tail)], out_hbm.at[pl.ds(base + s, tail)])


def _sc_gather(a0, a1, a2, T):
    E = a0.shape[0]
    assert E % NW == 0
    chunk = E // NW
    assert chunk % 8 == 0
    chunk_pad = ((chunk + 15) // 16) * 16
    mesh = plsc.VectorSubcoreMesh(core_axis_name="c", subcore_axis_name="s")
    kfn = pl.kernel(
        functools.partial(_sc_body, chunk),
        mesh=mesh,
        out_type=jax.ShapeDtypeStruct((E, HD), jnp.float32),
        scratch_types=[
            pltpu.VMEM((chunk_pad,), jnp.int32),
            pltpu.VMEM((chunk_pad,), jnp.int32),
            pltpu.VMEM((chunk_pad,), jnp.int32),
            pltpu.VMEM((chunk_pad,), jnp.int32),
            pltpu.VMEM((BLK, HD), jnp.float32),
            pltpu.SemaphoreType.DMA,
        ],
    )
    return kfn(a0, a1, a2, T)


def kernel(edge_attr, W0, W1, W2):
    T = _build_table(W0, W1, W2)
    a = edge_attr.astype(jnp.int32)
    a0 = a[:, 0]
    a1 = a[:, 1]
    a2 = a[:, 2]
    return _sc_gather(a0, a1, a2, T)
